# baseline (speedup 1.0000x reference)
import functools

import jax
import jax.numpy as jnp
from jax import lax
from jax.experimental import pallas as pl
from jax.experimental.pallas import tpu as pltpu
from jax.experimental.pallas import tpu_sc as plsc

L = 16


@functools.cache
def _build(B, T, S, V):
  NC, NS = 2, 16
  NW = NC * NS
  rows = B * T
  assert rows % NW == 0
  rpw = rows // NW
  assert (T % rpw == 0) and (S % L == 0) and (V % L == 0)
  ngrp = S // L

  mesh = plsc.VectorSubcoreMesh(core_axis_name="c", subcore_axis_name="s")

  @functools.partial(
      pl.kernel,
      out_type=jax.ShapeDtypeStruct((rows, V), jnp.float32),
      mesh=mesh,
      scratch_types=[
          pltpu.VMEM((V,), jnp.float32),
          pltpu.VMEM((V,), jnp.float32),
          pltpu.VMEM((rpw, S), jnp.float32),
          pltpu.VMEM((S,), jnp.int32),
          pltpu.SemaphoreType.DMA,
          pltpu.SemaphoreType.DMA,
      ],
      compiler_params=pltpu.CompilerParams(
          needs_layout_passes=False,
          use_tc_tiling_on_sc=False,
      ),
  )
  def scatter_rows(p_hbm, ids_hbm, out_hbm, buf0, buf1, p_v, ids_v, sem0, sem1):
    wid = lax.axis_index("s") * NC + lax.axis_index("c")
    base = wid * rpw
    batch = base // T

    pltpu.sync_copy(p_hbm.at[pl.ds(base, rpw)], p_v)
    pltpu.sync_copy(ids_hbm.at[batch], ids_v)

    zero = jnp.zeros((L,), jnp.float32)

    def zero_body(i, _):
      buf0[pl.ds(i * L, L)] = zero
      buf1[pl.ds(i * L, L)] = zero
      return 0

    lax.fori_loop(0, V // L, zero_body, 0)

    ivs = [ids_v[pl.ds(g * L, L)] for g in range(ngrp)]

    bufs = (buf0, buf1)
    sems = (sem0, sem1)
    pending = [None, None]

    for j in range(rpw):
      k = j % 2
      buf = bufs[k]
      if pending[k] is not None:
        pending[k].wait()
        for g in range(ngrp):
          plsc.store_scatter(buf, [ivs[g]], zero)
      for g in range(ngrp):
        plsc.addupdate_scatter(buf, [ivs[g]], p_v[j, pl.ds(g * L, L)])
      pending[k] = pltpu.async_copy(buf, out_hbm.at[base + j], sems[k])

    pending[0].wait()
    pending[1].wait()

  return scatter_rows


def kernel(p_source_position, p_target_vocab, input_source):
  B, T, S = p_source_position.shape
  V = p_target_vocab.shape[-1]
  fn = _build(B, T, S, V)
  p2 = p_source_position.reshape(B * T, S).astype(jnp.float32)
  out = fn(p2, input_source.astype(jnp.int32))
  return out.reshape(B, T, V)
